# baseline (speedup 1.0000x reference)
import functools

import jax
import jax.numpy as jnp
from jax import lax
from jax.experimental import pallas as pl
from jax.experimental.pallas import tpu as pltpu
from jax.experimental.pallas import tpu_sc as plsc

VOCAB = 1000000
DIM = 64
BATCH = 16384
BAG = 50

NC = 2
NS = 16
L = 16
NW = NC * NS
BPW = BATCH // NW
GROUP = 16
NGROUP = BPW // GROUP
ROWS = GROUP * BAG

PADW = 128
NBLK = VOCAB // PADW
BLK_BASE = NBLK // NW
BLK_EXTRA = NBLK % NW
TAILV = NBLK * PADW


def _mesh():
    return plsc.VectorSubcoreMesh(core_axis_name="c", subcore_axis_name="s")


def _transpose_block(in_v, out_v, iota, ncc):
    for k in range(DIM // L):
        for cc in range(ncc):
            col_ids = cc * L + iota
            for j in range(L):
                row = in_v[k * L + j, pl.ds(cc * L, L)]
                plsc.store_scatter(
                    out_v, [col_ids, jnp.full((L,), k * L + j, jnp.int32)], row
                )


def _sc_transpose_body(t_hbm, tail_hbm, scr_hbm, in_v, out_v):
    wid = lax.axis_index("s") * NC + lax.axis_index("c")
    nblk = BLK_BASE + jnp.where(wid < BLK_EXTRA, 1, 0)
    blk0 = BLK_BASE * wid + jnp.minimum(wid, BLK_EXTRA)
    iota = lax.iota(jnp.int32, L)

    def blk(i, carry):
        v0 = (blk0 + i) * PADW
        pltpu.sync_copy(t_hbm.at[pl.ds(0, DIM), pl.ds(v0, PADW)], in_v)
        _transpose_block(in_v, out_v, iota, PADW // L)
        pltpu.sync_copy(out_v, scr_hbm.at[pl.ds(v0, PADW)])
        return carry

    lax.fori_loop(0, nblk, blk, 0)

    @pl.when(wid == NW - 1)
    def _tail():
        pltpu.sync_copy(tail_hbm, in_v)
        _transpose_block(in_v, out_v, iota, PADW // L)
        pltpu.sync_copy(out_v, scr_hbm.at[pl.ds(VOCAB - PADW, PADW)])


def _sc_transpose(tT, tailT):
    f = functools.partial(
        pl.kernel,
        mesh=_mesh(),
        compiler_params=pltpu.CompilerParams(needs_layout_passes=False),
        out_type=jax.ShapeDtypeStruct((VOCAB, PADW), jnp.float32),
        scratch_types=[
            pltpu.VMEM((DIM, PADW), jnp.float32),
            pltpu.VMEM((PADW, PADW), jnp.float32),
        ],
    )(_sc_transpose_body)
    return f(tT, tailT)


def _sc_sum_body(idx_hbm, table_hbm, out_hbm, idx_v, rows_v, sums_v, sem):
    wid = lax.axis_index("s") * NC + lax.axis_index("c")
    base_bag = wid * BPW

    def group_body(g, carry):
        bag0 = base_bag + g * GROUP
        pltpu.sync_copy(idx_hbm.at[pl.ds(bag0, GROUP)], idx_v)
        copies = []
        for s in range(GROUP):
            copies.append(
                pltpu.async_copy(
                    table_hbm.at[idx_v.at[s]],
                    rows_v.at[pl.ds(s * BAG, BAG)],
                    sem,
                )
            )
        for c in copies:
            c.wait()
        for i in range(GROUP):
            row0 = i * BAG

            def bag_body(j, accs):
                base = row0 + j * 10
                for u in range(10):
                    accs = tuple(
                        accs[k] + rows_v[base + u, pl.ds(k * L, L)]
                        for k in range(DIM // L)
                    )
                return accs

            accs = lax.fori_loop(
                0, BAG // 10, bag_body,
                tuple(jnp.zeros((L,), jnp.float32) for _ in range(DIM // L)),
            )
            for k in range(DIM // L):
                sums_v[i, pl.ds(k * L, L)] = accs[k]
        pltpu.sync_copy(sums_v, out_hbm.at[pl.ds(bag0, GROUP)])
        return carry

    lax.fori_loop(0, NGROUP, group_body, 0)


def _sc_bag_sum(idx2d, scr):
    f = functools.partial(
        pl.kernel,
        mesh=_mesh(),
        out_type=jax.ShapeDtypeStruct((BATCH, DIM), jnp.float32),
        scratch_types=[
            pltpu.VMEM((GROUP, BAG), jnp.int32),
            pltpu.VMEM((ROWS, PADW), jnp.float32),
            pltpu.VMEM((GROUP, DIM), jnp.float32),
            pltpu.SemaphoreType.DMA,
        ],
    )(_sc_sum_body)
    return f(idx2d, scr)


def _tc_body(x_ref, w_ref, b_ref, o_ref):
    x = x_ref[...]
    n2 = jnp.sum(x * x, axis=1, keepdims=True)
    y = x * lax.rsqrt(n2)
    o_ref[...] = (
        lax.dot_general(y, w_ref[...], (((1,), (1,)), ((), ())),
                        preferred_element_type=jnp.float32)
        + b_ref[...]
    )


def _tc_norm_linear(sums, W, b2d):
    bs = 1024
    return pl.pallas_call(
        _tc_body,
        grid=(BATCH // bs,),
        in_specs=[
            pl.BlockSpec((bs, DIM), lambda i: (i, 0)),
            pl.BlockSpec((DIM, DIM), lambda i: (0, 0)),
            pl.BlockSpec((1, DIM), lambda i: (0, 0)),
        ],
        out_specs=pl.BlockSpec((bs, DIM), lambda i: (i, 0)),
        out_shape=jax.ShapeDtypeStruct((BATCH, DIM), jnp.float32),
    )(sums, W, b2d)


def kernel(token_idxs, table, W, b):
    tailT = lax.slice(table, (VOCAB - PADW, 0), (VOCAB, DIM)).T
    scr = _sc_transpose(table.T, tailT)
    sums = _sc_bag_sum(token_idxs.astype(jnp.int32), scr)
    return _tc_norm_linear(sums, W, b.reshape(1, DIM))


# speedup vs baseline: 1.8690x; 1.8690x over previous
import functools

import jax
import jax.numpy as jnp
from jax import lax
from jax.experimental import pallas as pl
from jax.experimental.pallas import tpu as pltpu
from jax.experimental.pallas import tpu_sc as plsc

VOCAB = 1000000
DIM = 64
BATCH = 16384
BAG = 50

NC = 2
NS = 16
L = 16
NW = NC * NS
BPW = BATCH // NW
GROUP = 16
NGROUP = BPW // GROUP
ROWS = GROUP * BAG

PADW = 128
NBLK = VOCAB // PADW
BLK_BASE = NBLK // NW
BLK_EXTRA = NBLK % NW
TAILV = NBLK * PADW


def _mesh():
    return plsc.VectorSubcoreMesh(core_axis_name="c", subcore_axis_name="s")


def _transpose_block(in_v, out_v, iota):
    def cc_body(cc, carry):
        cols = cc * L + iota
        for k in range(DIM // L):
            for t in range(L):
                rows = k * L + jnp.bitwise_and(iota + t, L - 1)
                v = plsc.load_gather(in_v, [rows, cols])
                plsc.store_scatter(out_v, [cols, rows], v)
        return carry

    lax.fori_loop(0, PADW // L, cc_body, 0)


def _sc_transpose_body(t_hbm, tail_hbm, scr_hbm, in_v2, out_v2,
                       si0, si1, so0, so1):
    wid = lax.axis_index("s") * NC + lax.axis_index("c")
    blk0 = wid * BLK_BASE
    iota = lax.iota(jnp.int32, L)
    sem_i = (si0, si1)
    sem_o = (so0, so1)

    def in_slice(b):
        return t_hbm.at[pl.ds(0, DIM), pl.ds((blk0 + b) * PADW, PADW)]

    def out_slice(b):
        return scr_hbm.at[pl.ds((blk0 + b) * PADW, PADW)]

    pltpu.async_copy(in_slice(0), in_v2.at[0], sem_i[0])

    def g_body(g, carry):
        for p in (0, 1):
            i = 2 * g + p
            nxt = i + 1

            @pl.when(nxt < BLK_BASE)
            def _start_next():
                pltpu.async_copy(in_slice(nxt), in_v2.at[1 - p], sem_i[1 - p])

            pltpu.make_async_copy(in_slice(0), in_v2.at[p], sem_i[p]).wait()

            @pl.when(g >= 1)
            def _drain_out():
                pltpu.make_async_copy(
                    out_v2.at[p], out_slice(0), sem_o[p]).wait()

            _transpose_block(in_v2.at[p], out_v2.at[p], iota)
            pltpu.async_copy(out_v2.at[p], out_slice(i), sem_o[p])
        return carry

    lax.fori_loop(0, BLK_BASE // 2, g_body, 0)
    for p in (0, 1):
        pltpu.make_async_copy(out_v2.at[p], out_slice(0), sem_o[p]).wait()

    for e in range(NBLK - NW * BLK_BASE):
        @pl.when(wid == e)
        def _extra():
            v0 = (NW * BLK_BASE + e) * PADW
            pltpu.sync_copy(t_hbm.at[pl.ds(0, DIM), pl.ds(v0, PADW)],
                            in_v2.at[0])
            _transpose_block(in_v2.at[0], out_v2.at[0], iota)
            pltpu.sync_copy(out_v2.at[0], scr_hbm.at[pl.ds(v0, PADW)])

    @pl.when(wid == 4)
    def _tail():
        pltpu.sync_copy(tail_hbm, in_v2.at[0])
        _transpose_block(in_v2.at[0], out_v2.at[0], iota)
        pltpu.sync_copy(
            out_v2.at[0, pl.ds(PADW - (VOCAB - TAILV), PADW // 2)],
            scr_hbm.at[pl.ds(TAILV, VOCAB - TAILV)],
        )


def _sc_transpose(tT, tailT):
    f = functools.partial(
        pl.kernel,
        mesh=_mesh(),
        compiler_params=pltpu.CompilerParams(needs_layout_passes=False),
        out_type=jax.ShapeDtypeStruct((VOCAB, PADW), jnp.float32),
        scratch_types=[
            pltpu.VMEM((2, DIM, PADW), jnp.float32),
            pltpu.VMEM((2, PADW, PADW), jnp.float32),
            pltpu.SemaphoreType.DMA,
            pltpu.SemaphoreType.DMA,
            pltpu.SemaphoreType.DMA,
            pltpu.SemaphoreType.DMA,
        ],
    )(_sc_transpose_body)
    return f(tT, tailT)


def _sc_sum_body(idx_hbm, table_hbm, out_hbm, idx_v, rows_v, sums_v, sem):
    wid = lax.axis_index("s") * NC + lax.axis_index("c")
    base_bag = wid * BPW

    def group_body(g, carry):
        bag0 = base_bag + g * GROUP
        pltpu.sync_copy(idx_hbm.at[pl.ds(bag0, GROUP)], idx_v)
        copies = []
        for s in range(GROUP):
            copies.append(
                pltpu.async_copy(
                    table_hbm.at[idx_v.at[s]],
                    rows_v.at[pl.ds(s * BAG, BAG)],
                    sem,
                )
            )
        for c in copies:
            c.wait()
        for i in range(GROUP):
            row0 = i * BAG

            def bag_body(j, accs):
                base = row0 + j * 10
                for u in range(10):
                    accs = tuple(
                        accs[k] + rows_v[base + u, pl.ds(k * L, L)]
                        for k in range(DIM // L)
                    )
                return accs

            accs = lax.fori_loop(
                0, BAG // 10, bag_body,
                tuple(jnp.zeros((L,), jnp.float32) for _ in range(DIM // L)),
            )
            for k in range(DIM // L):
                sums_v[i, pl.ds(k * L, L)] = accs[k]
        pltpu.sync_copy(sums_v, out_hbm.at[pl.ds(bag0, GROUP)])
        return carry

    lax.fori_loop(0, NGROUP, group_body, 0)


def _sc_bag_sum(idx2d, scr):
    f = functools.partial(
        pl.kernel,
        mesh=_mesh(),
        out_type=jax.ShapeDtypeStruct((BATCH, DIM), jnp.float32),
        scratch_types=[
            pltpu.VMEM((GROUP, BAG), jnp.int32),
            pltpu.VMEM((ROWS, PADW), jnp.float32),
            pltpu.VMEM((GROUP, DIM), jnp.float32),
            pltpu.SemaphoreType.DMA,
        ],
    )(_sc_sum_body)
    return f(idx2d, scr)


def _tc_body(x_ref, w_ref, b_ref, o_ref):
    x = x_ref[...]
    n2 = jnp.sum(x * x, axis=1, keepdims=True)
    y = x * lax.rsqrt(n2)
    o_ref[...] = (
        lax.dot_general(y, w_ref[...], (((1,), (1,)), ((), ())),
                        preferred_element_type=jnp.float32)
        + b_ref[...]
    )


def _tc_norm_linear(sums, W, b2d):
    bs = 1024
    return pl.pallas_call(
        _tc_body,
        grid=(BATCH // bs,),
        in_specs=[
            pl.BlockSpec((bs, DIM), lambda i: (i, 0)),
            pl.BlockSpec((DIM, DIM), lambda i: (0, 0)),
            pl.BlockSpec((1, DIM), lambda i: (0, 0)),
        ],
        out_specs=pl.BlockSpec((bs, DIM), lambda i: (i, 0)),
        out_shape=jax.ShapeDtypeStruct((BATCH, DIM), jnp.float32),
    )(sums, W, b2d)


def kernel(token_idxs, table, W, b):
    tailT = lax.slice(table, (VOCAB - PADW, 0), (VOCAB, DIM)).T
    scr = _sc_transpose(table.T, tailT)
    sums = _sc_bag_sum(token_idxs.astype(jnp.int32), scr)
    return _tc_norm_linear(sums, W, b.reshape(1, DIM))


# speedup vs baseline: 3.2260x; 1.7260x over previous
import functools

import jax
import jax.numpy as jnp
from jax import lax
from jax.experimental import pallas as pl
from jax.experimental.pallas import tpu as pltpu
from jax.experimental.pallas import tpu_sc as plsc

VOCAB = 1000000
DIM = 64
BATCH = 16384
BAG = 50

NC = 2
NS = 16
L = 16
NW = NC * NS
BPW = BATCH // NW
GROUP = 16
NGROUP = BPW // GROUP
ROWS = GROUP * BAG

PADW = 128
NBLK = VOCAB // PADW
BLK_BASE = NBLK // NW
BLK_EXTRA = NBLK % NW
TAILV = NBLK * PADW


def _mesh():
    return plsc.VectorSubcoreMesh(core_axis_name="c", subcore_axis_name="s")


def _transpose_block(in_v, out_v, iota):
    perms = [jnp.bitwise_and(iota + t, L - 1) for t in range(L)]

    def cc_body(cc, carry):
        cols = cc * L + iota
        for k in range(DIM // L):
            rows = [k * L + perms[t] for t in range(L)]
            vs = [plsc.load_gather(in_v, [rows[t], cols]) for t in range(L)]
            for t in range(L):
                plsc.store_scatter(out_v, [cols, rows[t]], vs[t])
        return carry

    lax.fori_loop(0, PADW // L, cc_body, 0)


def _sc_transpose_body(t_hbm, tail_hbm, scr_hbm, in_v2, out_v2,
                       si0, si1, so0, so1):
    wid = lax.axis_index("s") * NC + lax.axis_index("c")
    blk0 = wid * BLK_BASE
    iota = lax.iota(jnp.int32, L)
    sem_i = (si0, si1)
    sem_o = (so0, so1)

    def in_slice(b):
        return t_hbm.at[pl.ds(0, DIM), pl.ds((blk0 + b) * PADW, PADW)]

    def out_slice(b):
        return scr_hbm.at[pl.ds((blk0 + b) * PADW, PADW)]

    pltpu.async_copy(in_slice(0), in_v2.at[0], sem_i[0])

    def g_body(g, carry):
        for p in (0, 1):
            i = 2 * g + p
            nxt = i + 1

            @pl.when(nxt < BLK_BASE)
            def _start_next():
                pltpu.async_copy(in_slice(nxt), in_v2.at[1 - p], sem_i[1 - p])

            pltpu.make_async_copy(in_slice(0), in_v2.at[p], sem_i[p]).wait()

            @pl.when(g >= 1)
            def _drain_out():
                pltpu.make_async_copy(
                    out_v2.at[p], out_slice(0), sem_o[p]).wait()

            _transpose_block(in_v2.at[p], out_v2.at[p], iota)
            pltpu.async_copy(out_v2.at[p], out_slice(i), sem_o[p])
        return carry

    lax.fori_loop(0, BLK_BASE // 2, g_body, 0)
    for p in (0, 1):
        pltpu.make_async_copy(out_v2.at[p], out_slice(0), sem_o[p]).wait()

    for e in range(NBLK - NW * BLK_BASE):
        @pl.when(wid == e)
        def _extra():
            v0 = (NW * BLK_BASE + e) * PADW
            pltpu.sync_copy(t_hbm.at[pl.ds(0, DIM), pl.ds(v0, PADW)],
                            in_v2.at[0])
            _transpose_block(in_v2.at[0], out_v2.at[0], iota)
            pltpu.sync_copy(out_v2.at[0], scr_hbm.at[pl.ds(v0, PADW)])

    @pl.when(wid == 4)
    def _tail():
        pltpu.sync_copy(tail_hbm, in_v2.at[0])
        _transpose_block(in_v2.at[0], out_v2.at[0], iota)
        pltpu.sync_copy(
            out_v2.at[0, pl.ds(PADW - (VOCAB - TAILV), PADW // 2)],
            scr_hbm.at[pl.ds(TAILV, VOCAB - TAILV)],
        )


def _sc_transpose(tT, tailT):
    f = functools.partial(
        pl.kernel,
        mesh=_mesh(),
        compiler_params=pltpu.CompilerParams(needs_layout_passes=False),
        out_type=jax.ShapeDtypeStruct((VOCAB, PADW), jnp.float32),
        scratch_types=[
            pltpu.VMEM((2, DIM, PADW), jnp.float32),
            pltpu.VMEM((2, PADW, PADW), jnp.float32),
            pltpu.SemaphoreType.DMA,
            pltpu.SemaphoreType.DMA,
            pltpu.SemaphoreType.DMA,
            pltpu.SemaphoreType.DMA,
        ],
    )(_sc_transpose_body)
    return f(tT, tailT)


def _sc_sum_body(idx_hbm, table_hbm, out_hbm, idx_v, rows_v, sums_v, sem):
    wid = lax.axis_index("s") * NC + lax.axis_index("c")
    base_bag = wid * BPW

    def group_body(g, carry):
        bag0 = base_bag + g * GROUP
        pltpu.sync_copy(idx_hbm.at[pl.ds(bag0, GROUP)], idx_v)
        copies = []
        for s in range(GROUP):
            copies.append(
                pltpu.async_copy(
                    table_hbm.at[idx_v.at[s]],
                    rows_v.at[pl.ds(s * BAG, BAG)],
                    sem,
                )
            )
        for c in copies:
            c.wait()
        for i in range(GROUP):
            row0 = i * BAG

            def bag_body(j, accs):
                base = row0 + j * 10
                for u in range(10):
                    accs = tuple(
                        accs[k] + rows_v[base + u, pl.ds(k * L, L)]
                        for k in range(DIM // L)
                    )
                return accs

            accs = lax.fori_loop(
                0, BAG // 10, bag_body,
                tuple(jnp.zeros((L,), jnp.float32) for _ in range(DIM // L)),
            )
            for k in range(DIM // L):
                sums_v[i, pl.ds(k * L, L)] = accs[k]
        pltpu.sync_copy(sums_v, out_hbm.at[pl.ds(bag0, GROUP)])
        return carry

    lax.fori_loop(0, NGROUP, group_body, 0)


def _sc_bag_sum(idx2d, scr):
    f = functools.partial(
        pl.kernel,
        mesh=_mesh(),
        out_type=jax.ShapeDtypeStruct((BATCH, DIM), jnp.float32),
        scratch_types=[
            pltpu.VMEM((GROUP, BAG), jnp.int32),
            pltpu.VMEM((ROWS, PADW), jnp.float32),
            pltpu.VMEM((GROUP, DIM), jnp.float32),
            pltpu.SemaphoreType.DMA,
        ],
    )(_sc_sum_body)
    return f(idx2d, scr)


def _tc_body(x_ref, w_ref, b_ref, o_ref):
    x = x_ref[...]
    n2 = jnp.sum(x * x, axis=1, keepdims=True)
    y = x * lax.rsqrt(n2)
    o_ref[...] = (
        lax.dot_general(y, w_ref[...], (((1,), (1,)), ((), ())),
                        preferred_element_type=jnp.float32)
        + b_ref[...]
    )


def _tc_norm_linear(sums, W, b2d):
    bs = 1024
    return pl.pallas_call(
        _tc_body,
        grid=(BATCH // bs,),
        in_specs=[
            pl.BlockSpec((bs, DIM), lambda i: (i, 0)),
            pl.BlockSpec((DIM, DIM), lambda i: (0, 0)),
            pl.BlockSpec((1, DIM), lambda i: (0, 0)),
        ],
        out_specs=pl.BlockSpec((bs, DIM), lambda i: (i, 0)),
        out_shape=jax.ShapeDtypeStruct((BATCH, DIM), jnp.float32),
    )(sums, W, b2d)


def kernel(token_idxs, table, W, b):
    tailT = lax.slice(table, (VOCAB - PADW, 0), (VOCAB, DIM)).T
    scr = _sc_transpose(table.T, tailT)
    sums = _sc_bag_sum(token_idxs.astype(jnp.int32), scr)
    return _tc_norm_linear(sums, W, b.reshape(1, DIM))


# speedup vs baseline: 3.6028x; 1.1168x over previous
import functools

import jax
import jax.numpy as jnp
from jax import lax
from jax.experimental import pallas as pl
from jax.experimental.pallas import tpu as pltpu
from jax.experimental.pallas import tpu_sc as plsc

VOCAB = 1000000
DIM = 64
BATCH = 16384
BAG = 50

NC = 2
NS = 16
L = 16
NW = NC * NS
BPW = BATCH // NW
GROUP = 8
NGROUP = BPW // GROUP
ROWS = GROUP * BAG

PADW = 128
NBLK = VOCAB // PADW
BLK_BASE = NBLK // NW
BLK_EXTRA = NBLK % NW
TAILV = NBLK * PADW


def _mesh():
    return plsc.VectorSubcoreMesh(core_axis_name="c", subcore_axis_name="s")


def _transpose_block(in_v, out_v, iota):
    perms = [jnp.bitwise_and(iota + t, L - 1) for t in range(L)]

    def cc_body(cc, carry):
        cols = cc * L + iota
        for k in range(DIM // L):
            rows = [k * L + perms[t] for t in range(L)]
            vs = [plsc.load_gather(in_v, [rows[t], cols]) for t in range(L)]
            for t in range(L):
                plsc.store_scatter(out_v, [cols, rows[t]], vs[t])
        return carry

    lax.fori_loop(0, PADW // L, cc_body, 0)


def _sc_transpose_body(t_hbm, tail_hbm, scr_hbm, in_v2, out_v2,
                       si0, si1, so0, so1):
    wid = lax.axis_index("s") * NC + lax.axis_index("c")
    blk0 = wid * BLK_BASE
    iota = lax.iota(jnp.int32, L)
    sem_i = (si0, si1)
    sem_o = (so0, so1)

    def in_slice(b):
        return t_hbm.at[pl.ds(0, DIM), pl.ds((blk0 + b) * PADW, PADW)]

    def out_slice(b):
        return scr_hbm.at[pl.ds((blk0 + b) * PADW, PADW)]

    pltpu.async_copy(in_slice(0), in_v2.at[0], sem_i[0])

    def g_body(g, carry):
        for p in (0, 1):
            i = 2 * g + p
            nxt = i + 1

            @pl.when(nxt < BLK_BASE)
            def _start_next():
                pltpu.async_copy(in_slice(nxt), in_v2.at[1 - p], sem_i[1 - p])

            pltpu.make_async_copy(in_slice(0), in_v2.at[p], sem_i[p]).wait()

            @pl.when(g >= 1)
            def _drain_out():
                pltpu.make_async_copy(
                    out_v2.at[p], out_slice(0), sem_o[p]).wait()

            _transpose_block(in_v2.at[p], out_v2.at[p], iota)
            pltpu.async_copy(out_v2.at[p], out_slice(i), sem_o[p])
        return carry

    lax.fori_loop(0, BLK_BASE // 2, g_body, 0)
    for p in (0, 1):
        pltpu.make_async_copy(out_v2.at[p], out_slice(0), sem_o[p]).wait()

    for e in range(NBLK - NW * BLK_BASE):
        @pl.when(wid == e)
        def _extra():
            v0 = (NW * BLK_BASE + e) * PADW
            pltpu.sync_copy(t_hbm.at[pl.ds(0, DIM), pl.ds(v0, PADW)],
                            in_v2.at[0])
            _transpose_block(in_v2.at[0], out_v2.at[0], iota)
            pltpu.sync_copy(out_v2.at[0], scr_hbm.at[pl.ds(v0, PADW)])

    @pl.when(wid == 4)
    def _tail():
        pltpu.sync_copy(tail_hbm, in_v2.at[0])
        _transpose_block(in_v2.at[0], out_v2.at[0], iota)
        pltpu.sync_copy(
            out_v2.at[0, pl.ds(PADW - (VOCAB - TAILV), PADW // 2)],
            scr_hbm.at[pl.ds(TAILV, VOCAB - TAILV)],
        )


def _sc_transpose(tT, tailT):
    f = functools.partial(
        pl.kernel,
        mesh=_mesh(),
        compiler_params=pltpu.CompilerParams(needs_layout_passes=False),
        out_type=jax.ShapeDtypeStruct((VOCAB, PADW), jnp.float32),
        scratch_types=[
            pltpu.VMEM((2, DIM, PADW), jnp.float32),
            pltpu.VMEM((2, PADW, PADW), jnp.float32),
            pltpu.SemaphoreType.DMA,
            pltpu.SemaphoreType.DMA,
            pltpu.SemaphoreType.DMA,
            pltpu.SemaphoreType.DMA,
        ],
    )(_sc_transpose_body)
    return f(tT, tailT)


def _sc_sum_body(idx_hbm, table_hbm, out_hbm, idx_v2, rows_v2, sums_v2,
                 mi0, mi1, mr0, mr1, mo0, mo1):
    wid = lax.axis_index("s") * NC + lax.axis_index("c")
    base_bag = wid * BPW
    sem_i = (mi0, mi1)
    sem_r = (mr0, mr1)
    sem_o = (mo0, mo1)

    def idx_slice(g):
        return idx_hbm.at[pl.ds(base_bag + g * GROUP, GROUP)]

    def out_slice(g):
        return out_hbm.at[pl.ds(base_bag + g * GROUP, GROUP)]

    def fire_gathers(p):
        for s in range(GROUP):
            pltpu.async_copy(
                table_hbm.at[idx_v2.at[p].at[s]],
                rows_v2.at[p].at[pl.ds(s * BAG, BAG)],
                sem_r[p],
            )

    pltpu.sync_copy(idx_slice(0), idx_v2.at[0])
    fire_gathers(0)
    pltpu.async_copy(idx_slice(1), idx_v2.at[1], sem_i[1])

    def g_body(g, carry):
        for p in (0, 1):
            i = 2 * g + p

            @pl.when(i + 1 < NGROUP)
            def _next_gathers():
                pltpu.make_async_copy(
                    idx_slice(0), idx_v2.at[1 - p], sem_i[1 - p]).wait()
                fire_gathers(1 - p)

            pltpu.make_async_copy(
                table_hbm.at[pl.ds(0, ROWS)], rows_v2.at[p], sem_r[p]).wait()

            @pl.when(i + 2 < NGROUP)
            def _next_idx():
                pltpu.async_copy(idx_slice(i + 2), idx_v2.at[p], sem_i[p])

            @pl.when(i >= 2)
            def _drain_out():
                pltpu.make_async_copy(
                    sums_v2.at[p], out_slice(0), sem_o[p]).wait()

            for b in range(GROUP):
                row0 = b * BAG

                def bag_body(j, accs):
                    base = row0 + j * 10
                    for u in range(10):
                        accs = tuple(
                            accs[k] + rows_v2[p, base + u, pl.ds(k * L, L)]
                            for k in range(DIM // L)
                        )
                    return accs

                accs = lax.fori_loop(
                    0, BAG // 10, bag_body,
                    tuple(jnp.zeros((L,), jnp.float32)
                          for _ in range(DIM // L)),
                )
                for k in range(DIM // L):
                    sums_v2[p, b, pl.ds(k * L, L)] = accs[k]
            pltpu.async_copy(sums_v2.at[p], out_slice(i), sem_o[p])
        return carry

    lax.fori_loop(0, NGROUP // 2, g_body, 0)
    for p in (0, 1):
        pltpu.make_async_copy(sums_v2.at[p], out_slice(0), sem_o[p]).wait()


def _sc_bag_sum(idx2d, scr):
    f = functools.partial(
        pl.kernel,
        mesh=_mesh(),
        out_type=jax.ShapeDtypeStruct((BATCH, DIM), jnp.float32),
        scratch_types=[
            pltpu.VMEM((2, GROUP, BAG), jnp.int32),
            pltpu.VMEM((2, ROWS, PADW), jnp.float32),
            pltpu.VMEM((2, GROUP, DIM), jnp.float32),
            pltpu.SemaphoreType.DMA,
            pltpu.SemaphoreType.DMA,
            pltpu.SemaphoreType.DMA,
            pltpu.SemaphoreType.DMA,
            pltpu.SemaphoreType.DMA,
            pltpu.SemaphoreType.DMA,
        ],
    )(_sc_sum_body)
    return f(idx2d, scr)


def _tc_body(x_ref, w_ref, b_ref, o_ref):
    x = x_ref[...]
    n2 = jnp.sum(x * x, axis=1, keepdims=True)
    y = x * lax.rsqrt(n2)
    o_ref[...] = (
        lax.dot_general(y, w_ref[...], (((1,), (1,)), ((), ())),
                        preferred_element_type=jnp.float32)
        + b_ref[...]
    )


def _tc_norm_linear(sums, W, b2d):
    bs = 1024
    return pl.pallas_call(
        _tc_body,
        grid=(BATCH // bs,),
        in_specs=[
            pl.BlockSpec((bs, DIM), lambda i: (i, 0)),
            pl.BlockSpec((DIM, DIM), lambda i: (0, 0)),
            pl.BlockSpec((1, DIM), lambda i: (0, 0)),
        ],
        out_specs=pl.BlockSpec((bs, DIM), lambda i: (i, 0)),
        out_shape=jax.ShapeDtypeStruct((BATCH, DIM), jnp.float32),
    )(sums, W, b2d)


def kernel(token_idxs, table, W, b):
    tailT = lax.slice(table, (VOCAB - PADW, 0), (VOCAB, DIM)).T
    scr = _sc_transpose(table.T, tailT)
    sums = _sc_bag_sum(token_idxs.astype(jnp.int32), scr)
    return _tc_norm_linear(sums, W, b.reshape(1, DIM))


# speedup vs baseline: 3.6614x; 1.0163x over previous
import functools

import jax
import jax.numpy as jnp
from jax import lax
from jax.experimental import pallas as pl
from jax.experimental.pallas import tpu as pltpu
from jax.experimental.pallas import tpu_sc as plsc

VOCAB = 1000000
DIM = 64
BATCH = 16384
BAG = 50

NC = 2
NS = 16
L = 16
NW = NC * NS
BPW = BATCH // NW
GROUP = 8
NGROUP = BPW // GROUP
ROWS = GROUP * BAG

PADW = 128
BLKW = 256
NBLK = VOCAB // BLKW
BLK_BASE = NBLK // NW
BLK_EXTRA = NBLK - NW * BLK_BASE
TAILV = NBLK * BLKW


def _mesh():
    return plsc.VectorSubcoreMesh(core_axis_name="c", subcore_axis_name="s")


def _transpose_block(in_v, out_v, iota):
    perms = [jnp.bitwise_and(iota + t, L - 1) for t in range(L)]

    def cc_body(cc, carry):
        cols = cc * L + iota
        for k in range(DIM // L):
            rows = [k * L + perms[t] for t in range(L)]
            vs = [plsc.load_gather(in_v, [rows[t], cols]) for t in range(L)]
            for t in range(L):
                plsc.store_scatter(out_v, [cols, rows[t]], vs[t])
        return carry

    lax.fori_loop(0, in_v.shape[1] // L, cc_body, 0)


def _sc_transpose_body(t_hbm, tail_hbm, scr_hbm, in_v2, out_v2, tail_v,
                       si0, si1, so0, so1):
    wid = lax.axis_index("s") * NC + lax.axis_index("c")
    blk0 = wid * BLK_BASE
    iota = lax.iota(jnp.int32, L)
    sem_i = (si0, si1)
    sem_o = (so0, so1)

    def in_slice(b):
        return t_hbm.at[pl.ds(0, DIM), pl.ds((blk0 + b) * BLKW, BLKW)]

    def out_slice(b):
        return scr_hbm.at[pl.ds((blk0 + b) * BLKW, BLKW)]

    pltpu.async_copy(in_slice(0), in_v2.at[0], sem_i[0])

    def g_body(g, carry):
        for p in (0, 1):
            i = 2 * g + p
            nxt = i + 1

            @pl.when(nxt < BLK_BASE)
            def _start_next():
                pltpu.async_copy(in_slice(nxt), in_v2.at[1 - p], sem_i[1 - p])

            pltpu.make_async_copy(in_slice(0), in_v2.at[p], sem_i[p]).wait()

            @pl.when(g >= 1)
            def _drain_out():
                pltpu.make_async_copy(
                    out_v2.at[p], out_slice(0), sem_o[p]).wait()

            _transpose_block(in_v2.at[p], out_v2.at[p], iota)
            pltpu.async_copy(out_v2.at[p], out_slice(i), sem_o[p])
        return carry

    lax.fori_loop(0, BLK_BASE // 2, g_body, 0)
    for p in (0, 1):
        pltpu.make_async_copy(out_v2.at[p], out_slice(0), sem_o[p]).wait()

    for e in range(BLK_EXTRA):
        @pl.when(wid == e)
        def _extra():
            v0 = (NW * BLK_BASE + e) * BLKW
            pltpu.sync_copy(t_hbm.at[pl.ds(0, DIM), pl.ds(v0, BLKW)],
                            in_v2.at[0])
            _transpose_block(in_v2.at[0], out_v2.at[0], iota)
            pltpu.sync_copy(out_v2.at[0], scr_hbm.at[pl.ds(v0, BLKW)])

    @pl.when(wid == 4)
    def _tail():
        pltpu.sync_copy(tail_hbm, tail_v)
        _transpose_block(tail_v, out_v2.at[0], iota)
        pltpu.sync_copy(
            out_v2.at[0, pl.ds(PADW - (VOCAB - TAILV), PADW // 2)],
            scr_hbm.at[pl.ds(TAILV, VOCAB - TAILV)],
        )


def _sc_transpose(tT, tailT):
    f = functools.partial(
        pl.kernel,
        mesh=_mesh(),
        compiler_params=pltpu.CompilerParams(needs_layout_passes=False),
        out_type=jax.ShapeDtypeStruct((VOCAB, PADW), jnp.float32),
        scratch_types=[
            pltpu.VMEM((2, DIM, BLKW), jnp.float32),
            pltpu.VMEM((2, BLKW, PADW), jnp.float32),
            pltpu.VMEM((DIM, PADW), jnp.float32),
            pltpu.SemaphoreType.DMA,
            pltpu.SemaphoreType.DMA,
            pltpu.SemaphoreType.DMA,
            pltpu.SemaphoreType.DMA,
        ],
    )(_sc_transpose_body)
    return f(tT, tailT)


def _sc_sum_body(idx_hbm, table_hbm, out_hbm, idx_v2, rows_v2, sums_v2,
                 mi0, mi1, mr0, mr1, mo0, mo1):
    wid = lax.axis_index("s") * NC + lax.axis_index("c")
    base_bag = wid * BPW
    sem_i = (mi0, mi1)
    sem_r = (mr0, mr1)
    sem_o = (mo0, mo1)

    def idx_slice(g):
        return idx_hbm.at[pl.ds(base_bag + g * GROUP, GROUP)]

    def out_slice(g):
        return out_hbm.at[pl.ds(base_bag + g * GROUP, GROUP)]

    def fire_gathers(p):
        for s in range(GROUP):
            pltpu.async_copy(
                table_hbm.at[idx_v2.at[p].at[s]],
                rows_v2.at[p].at[pl.ds(s * BAG, BAG)],
                sem_r[p],
            )

    pltpu.sync_copy(idx_slice(0), idx_v2.at[0])
    fire_gathers(0)
    pltpu.async_copy(idx_slice(1), idx_v2.at[1], sem_i[1])

    def g_body(g, carry):
        for p in (0, 1):
            i = 2 * g + p

            @pl.when(i + 1 < NGROUP)
            def _next_gathers():
                pltpu.make_async_copy(
                    idx_slice(0), idx_v2.at[1 - p], sem_i[1 - p]).wait()
                fire_gathers(1 - p)

            pltpu.make_async_copy(
                table_hbm.at[pl.ds(0, ROWS)], rows_v2.at[p], sem_r[p]).wait()

            @pl.when(i + 2 < NGROUP)
            def _next_idx():
                pltpu.async_copy(idx_slice(i + 2), idx_v2.at[p], sem_i[p])

            @pl.when(i >= 2)
            def _drain_out():
                pltpu.make_async_copy(
                    sums_v2.at[p], out_slice(0), sem_o[p]).wait()

            for b in range(GROUP):
                row0 = b * BAG

                def bag_body(j, accs):
                    base = row0 + j * 10
                    for u in range(10):
                        accs = tuple(
                            accs[k] + rows_v2[p, base + u, pl.ds(k * L, L)]
                            for k in range(DIM // L)
                        )
                    return accs

                accs = lax.fori_loop(
                    0, BAG // 10, bag_body,
                    tuple(jnp.zeros((L,), jnp.float32)
                          for _ in range(DIM // L)),
                )
                for k in range(DIM // L):
                    sums_v2[p, b, pl.ds(k * L, L)] = accs[k]
            pltpu.async_copy(sums_v2.at[p], out_slice(i), sem_o[p])
        return carry

    lax.fori_loop(0, NGROUP // 2, g_body, 0)
    for p in (0, 1):
        pltpu.make_async_copy(sums_v2.at[p], out_slice(0), sem_o[p]).wait()


def _sc_bag_sum(idx2d, scr):
    f = functools.partial(
        pl.kernel,
        mesh=_mesh(),
        out_type=jax.ShapeDtypeStruct((BATCH, DIM), jnp.float32),
        scratch_types=[
            pltpu.VMEM((2, GROUP, BAG), jnp.int32),
            pltpu.VMEM((2, ROWS, PADW), jnp.float32),
            pltpu.VMEM((2, GROUP, DIM), jnp.float32),
            pltpu.SemaphoreType.DMA,
            pltpu.SemaphoreType.DMA,
            pltpu.SemaphoreType.DMA,
            pltpu.SemaphoreType.DMA,
            pltpu.SemaphoreType.DMA,
            pltpu.SemaphoreType.DMA,
        ],
    )(_sc_sum_body)
    return f(idx2d, scr)


def _tc_body(x_ref, w_ref, b_ref, o_ref):
    x = x_ref[...]
    n2 = jnp.sum(x * x, axis=1, keepdims=True)
    y = x * lax.rsqrt(n2)
    o_ref[...] = (
        lax.dot_general(y, w_ref[...], (((1,), (1,)), ((), ())),
                        preferred_element_type=jnp.float32)
        + b_ref[...]
    )


def _tc_norm_linear(sums, W, b2d):
    bs = 1024
    return pl.pallas_call(
        _tc_body,
        grid=(BATCH // bs,),
        in_specs=[
            pl.BlockSpec((bs, DIM), lambda i: (i, 0)),
            pl.BlockSpec((DIM, DIM), lambda i: (0, 0)),
            pl.BlockSpec((1, DIM), lambda i: (0, 0)),
        ],
        out_specs=pl.BlockSpec((bs, DIM), lambda i: (i, 0)),
        out_shape=jax.ShapeDtypeStruct((BATCH, DIM), jnp.float32),
    )(sums, W, b2d)


def kernel(token_idxs, table, W, b):
    tailT = lax.slice(table, (VOCAB - PADW, 0), (VOCAB, DIM)).T
    scr = _sc_transpose(table.T, tailT)
    sums = _sc_bag_sum(token_idxs.astype(jnp.int32), scr)
    return _tc_norm_linear(sums, W, b.reshape(1, DIM))
